# baseline (speedup 1.0000x reference)
import functools

import jax
import jax.numpy as jnp
from jax import lax
from jax.experimental import pallas as pl
from jax.experimental.pallas import tpu as pltpu
from jax.experimental.pallas import tpu_sc as plsc

B = 16384
F = 26
V = 100000
D = 16

_INFO = plsc.get_sparse_core_info()
NC = _INFO.num_cores
NS = _INFO.num_subcores
L = _INFO.num_lanes
NW = NC * NS

RW = B // NW

_mesh = plsc.VectorSubcoreMesh(core_axis_name="c", subcore_axis_name="s")


@functools.partial(
    pl.kernel,
    mesh=_mesh,
    out_type=jax.ShapeDtypeStruct((B, F * D), jnp.float32),
    scratch_types=[
        pltpu.VMEM((RW, F), jnp.int32),
        pltpu.VMEM((RW,), jnp.int32),
        pltpu.VMEM((RW, D), jnp.float32),
        pltpu.SemaphoreType.DMA,
    ],
    compiler_params=pltpu.CompilerParams(
        use_tc_tiling_on_sc=False, needs_layout_passes=False
    ),
)
def _gather_rows(table_hbm, idx_hbm, out_hbm, idx_v, ids_v, rows_v, sem):
    wid = lax.axis_index("s") * NC + lax.axis_index("c")
    b0 = wid * RW

    pltpu.sync_copy(idx_hbm.at[pl.ds(b0, RW), :], idx_v)

    lane = lax.iota(jnp.int32, L)

    def f_body(fi, _):
        col = jnp.full((L,), fi, jnp.int32)

        def ext(j, _):
            ids_v[pl.ds(j * L, L)] = plsc.load_gather(idx_v, [lane + j * L, col])
            return ()

        lax.fori_loop(0, RW // L, ext, ())

        pltpu.async_copy(table_hbm.at[fi].at[ids_v], rows_v, sem).wait()

        pltpu.sync_copy(rows_v, out_hbm.at[pl.ds(b0, RW), pl.ds(fi * D, D)])
        return ()

    lax.fori_loop(0, F, f_body, ())


def kernel(indices, tables):
    return _gather_rows(tables, indices)


# speedup vs baseline: 1.0245x; 1.0245x over previous
import functools

import jax
import jax.numpy as jnp
from jax import lax
from jax.experimental import pallas as pl
from jax.experimental.pallas import tpu as pltpu
from jax.experimental.pallas import tpu_sc as plsc

B = 16384
F = 26
V = 100000
D = 16

_INFO = plsc.get_sparse_core_info()
NC = _INFO.num_cores
NS = _INFO.num_subcores
L = _INFO.num_lanes
NW = NC * NS

RW = B // NW
RC = 128
NCH = RW // RC
CN = RC * F

_mesh = plsc.VectorSubcoreMesh(core_axis_name="c", subcore_axis_name="s")


@functools.partial(
    pl.kernel,
    mesh=_mesh,
    out_type=jax.ShapeDtypeStruct((B * F, D), jnp.float32),
    scratch_types=[
        pltpu.VMEM((RC, F), jnp.int32),
        pltpu.VMEM((CN,), jnp.int32),
        pltpu.VMEM((CN, D), jnp.float32),
        pltpu.SemaphoreType.DMA,
    ],
    compiler_params=pltpu.CompilerParams(
        use_tc_tiling_on_sc=False, needs_layout_passes=False
    ),
)
def _gather_rows(table_hbm, idx_hbm, out_hbm, idx_v, ids_v, rows_v, sem):
    wid = lax.axis_index("s") * NC + lax.axis_index("c")
    row0 = wid * RW

    off_lo = lax.iota(jnp.int32, L) * V
    off_hi = (lax.iota(jnp.int32, L) + (F - L)) * V

    def chunk_body(i, _):
        b0 = row0 + i * RC
        pltpu.sync_copy(idx_hbm.at[pl.ds(b0, RC), :], idx_v)

        def row_body(r, _):
            ids_v[pl.ds(r * F, L)] = idx_v[r, pl.ds(0, L)] + off_lo
            ids_v[pl.ds(r * F + (F - L), L)] = idx_v[r, pl.ds(F - L, L)] + off_hi
            return ()

        lax.fori_loop(0, RC, row_body, ())

        pltpu.async_copy(table_hbm.at[ids_v], rows_v, sem).wait()

        pltpu.sync_copy(rows_v, out_hbm.at[pl.ds(b0 * F, CN)])
        return ()

    lax.fori_loop(0, NCH, chunk_body, ())


def kernel(indices, tables):
    wide = jax.lax.optimization_barrier(tables.reshape(F * V // 8, 8 * D))
    out = _gather_rows(wide.reshape(F * V, D), indices)
    return out.reshape(B, F * D)
